# baseline (device time: 21279 ns/iter reference)
import jax
import jax.numpy as jnp
from jax import lax
from jax.experimental import pallas as pl
from jax.experimental.pallas import tpu as pltpu

N_DEV = 4
B, Sq, Skv = 2, 128, 128
HQ_LOCAL, Dh = 4, 64
HD = HQ_LOCAL * Dh
D_MODEL = 512
D_OUT = 512


def kernel(x, Wq, K_ext, V_ext, Wo):
    def body(x_ref, wq_ref, k_ref, v_ref, wo_ref, out_ref,
             ctx_ref, send_sems, recv_sems):
        my = lax.axis_index("i")

        barrier_sem = pltpu.get_barrier_semaphore()
        for off in range(1, N_DEV):
            pl.semaphore_signal(
                barrier_sem, inc=1,
                device_id=((my + off) % N_DEV,),
                device_id_type=pl.DeviceIdType.MESH,
            )
        pl.semaphore_wait(barrier_sem, N_DEV - 1)

        x2 = x_ref[...].reshape(B * Sq, D_MODEL)
        wq_slice = wq_ref[:, pl.ds(my * HD, HD)]
        q = jnp.dot(x2, wq_slice, preferred_element_type=jnp.float32)
        q = q.reshape(B, Sq, HQ_LOCAL, Dh)

        qb = lax.broadcasted_iota(jnp.int32, (Sq, Skv), 0) // 64
        kb = lax.broadcasted_iota(jnp.int32, (Sq, Skv), 1) // 64
        mask = (qb == kb) | (kb == 0) | ((qb + kb) % 3 == 0)

        k = k_ref[...]
        v = v_ref[...]
        ctx_rows = []
        for b in range(B):
            heads = []
            for h in range(HQ_LOCAL):
                s = lax.dot_general(
                    q[b, :, h, :], k[b, :, h, :],
                    (((1,), (1,)), ((), ())),
                    preferred_element_type=jnp.float32,
                ) * 0.125
                s = jnp.where(mask, s, -1e9)
                m = jnp.max(s, axis=-1, keepdims=True)
                w = jnp.exp(s - m)
                w = w / jnp.sum(w, axis=-1, keepdims=True)
                heads.append(jnp.dot(w, v[b, :, h, :],
                                     preferred_element_type=jnp.float32))
            ctx_rows.append(jnp.concatenate(heads, axis=1))
        ctx_ref[0] = jnp.concatenate(ctx_rows, axis=0)

        sends = []
        for off in range(1, N_DEV):
            dst_slot = N_DEV - off
            rdma = pltpu.make_async_remote_copy(
                src_ref=ctx_ref.at[0],
                dst_ref=ctx_ref.at[dst_slot],
                send_sem=send_sems.at[off - 1],
                recv_sem=recv_sems.at[dst_slot - 1],
                device_id=((my + off) % N_DEV,),
                device_id_type=pl.DeviceIdType.MESH,
            )
            rdma.start()
            sends.append(rdma)

        acc = jnp.dot(ctx_ref[0], wo_ref[pl.ds(my * HD, HD), :],
                      preferred_element_type=jnp.float32)
        for j in range(1, N_DEV):
            recv = pltpu.make_async_remote_copy(
                src_ref=ctx_ref.at[j],
                dst_ref=ctx_ref.at[j],
                send_sem=send_sems.at[j - 1],
                recv_sem=recv_sems.at[j - 1],
                device_id=((my + j) % N_DEV,),
                device_id_type=pl.DeviceIdType.MESH,
            )
            recv.wait_recv()
            src_dev = (my + j) % N_DEV
            acc = acc + jnp.dot(ctx_ref[j],
                                wo_ref[pl.ds(src_dev * HD, HD), :],
                                preferred_element_type=jnp.float32)
        out_ref[...] = acc.reshape(B, Sq, D_OUT)

        for rdma in sends:
            rdma.wait_send()

    return pl.pallas_call(
        body,
        out_shape=jax.ShapeDtypeStruct((B, Sq, D_OUT), jnp.float32),
        in_specs=[pl.BlockSpec(memory_space=pltpu.VMEM)] * 5,
        out_specs=pl.BlockSpec(memory_space=pltpu.VMEM),
        scratch_shapes=[
            pltpu.VMEM((N_DEV, B * Sq, HD), jnp.float32),
            pltpu.SemaphoreType.DMA((N_DEV - 1,)),
            pltpu.SemaphoreType.DMA((N_DEV - 1,)),
        ],
        compiler_params=pltpu.CompilerParams(collective_id=0),
    )(x, Wq, K_ext, V_ext, Wo)


# device time: 18168 ns/iter; 1.1712x vs baseline; 1.1712x over previous
import jax
import jax.numpy as jnp
from jax import lax
from jax.experimental import pallas as pl
from jax.experimental.pallas import tpu as pltpu

N_DEV = 4
B, Sq, Skv = 2, 128, 128
BS = B * Sq
HQ_LOCAL, Dh = 4, 64
HD = HQ_LOCAL * Dh
D_MODEL = 512
D_OUT = 512


def kernel(x, Wq, K_ext, V_ext, Wo):
    my_out = lax.axis_index("i")
    Kt = jnp.transpose(K_ext, (0, 2, 3, 1))
    Wq_sl = lax.dynamic_slice(Wq, (0, my_out * HD), (D_MODEL, HD))

    def body(x_ref, wq_ref, kt_ref, v_ref, wo_ref, out_ref,
             ctx_ref, send_sems, recv_sems):
        my = lax.axis_index("i")

        x2 = x_ref[...].reshape(BS, D_MODEL)
        q = jnp.dot(x2, wq_ref[...], preferred_element_type=jnp.float32)

        barrier_sem = pltpu.get_barrier_semaphore()
        for off in range(1, N_DEV):
            pl.semaphore_signal(
                barrier_sem, inc=1,
                device_id=((my + off) % N_DEV,),
                device_id_type=pl.DeviceIdType.MESH,
            )
        pl.semaphore_wait(barrier_sem, N_DEV - 1)

        row = lax.broadcasted_iota(jnp.int32, (BS, BS), 0)
        col = lax.broadcasted_iota(jnp.int32, (BS, BS), 1)
        qb = (row % Sq) // 64
        kb = (col % Skv) // 64
        mask = ((row // Sq) == (col // Skv)) & (
            (qb == kb) | (kb == 0) | ((qb + kb) % 3 == 0)
        )
        neg = jnp.float32(-1e9)

        v2 = v_ref[...].reshape(BS, HD)
        heads = []
        for h in range(HQ_LOCAL):
            kT = jnp.concatenate(
                [kt_ref[0, h], kt_ref[1, h]], axis=1)
            s = jnp.dot(q[:, h * Dh:(h + 1) * Dh], kT,
                        preferred_element_type=jnp.float32) * 0.125
            s = jnp.where(mask, s, neg)
            w = jnp.exp(s - jnp.max(s, axis=-1, keepdims=True))
            w = w / jnp.sum(w, axis=-1, keepdims=True)
            heads.append(jnp.dot(w, v2[:, h * Dh:(h + 1) * Dh],
                                 preferred_element_type=jnp.float32))
        ctx_ref[0] = jnp.concatenate(heads, axis=1)

        sends = []
        for off in (2, 1, 3):
            dst_slot = N_DEV - off
            rdma = pltpu.make_async_remote_copy(
                src_ref=ctx_ref.at[0],
                dst_ref=ctx_ref.at[dst_slot],
                send_sem=send_sems.at[off - 1],
                recv_sem=recv_sems.at[dst_slot - 1],
                device_id=((my + off) % N_DEV,),
                device_id_type=pl.DeviceIdType.MESH,
            )
            rdma.start()
            sends.append(rdma)

        acc = jnp.dot(ctx_ref[0], wo_ref[pl.ds(my * HD, HD), :],
                      preferred_element_type=jnp.float32)
        for j in (1, 3, 2):
            recv = pltpu.make_async_remote_copy(
                src_ref=ctx_ref.at[j],
                dst_ref=ctx_ref.at[j],
                send_sem=send_sems.at[j - 1],
                recv_sem=recv_sems.at[j - 1],
                device_id=((my + j) % N_DEV,),
                device_id_type=pl.DeviceIdType.MESH,
            )
            recv.wait_recv()
            src_dev = (my + j) % N_DEV
            acc = acc + jnp.dot(ctx_ref[j],
                                wo_ref[pl.ds(src_dev * HD, HD), :],
                                preferred_element_type=jnp.float32)
        out_ref[...] = acc.reshape(B, Sq, D_OUT)

        for rdma in sends:
            rdma.wait_send()

    return pl.pallas_call(
        body,
        out_shape=jax.ShapeDtypeStruct((B, Sq, D_OUT), jnp.float32),
        in_specs=[pl.BlockSpec(memory_space=pltpu.VMEM)] * 5,
        out_specs=pl.BlockSpec(memory_space=pltpu.VMEM),
        scratch_shapes=[
            pltpu.VMEM((N_DEV, BS, HD), jnp.float32),
            pltpu.SemaphoreType.DMA((N_DEV - 1,)),
            pltpu.SemaphoreType.DMA((N_DEV - 1,)),
        ],
        compiler_params=pltpu.CompilerParams(collective_id=0),
    )(x, Wq_sl, Kt, V_ext, Wo)


# device time: 12345 ns/iter; 1.7237x vs baseline; 1.4717x over previous
import jax
import jax.numpy as jnp
from jax import lax
from jax.experimental import pallas as pl
from jax.experimental.pallas import tpu as pltpu

N_DEV = 4
B, Sq, Skv = 2, 128, 128
BS = B * Sq
HQ_LOCAL, Dh = 4, 64
HD = HQ_LOCAL * Dh
D_MODEL = 512
D_OUT = 512


def kernel(x, Wq, K_ext, V_ext, Wo):
    my_out = lax.axis_index("i")
    Wq_sl = lax.dynamic_slice(Wq, (0, my_out * HD), (D_MODEL, HD))
    Kt = jnp.transpose(K_ext, (0, 2, 3, 1))
    Vt = jnp.transpose(V_ext, (0, 2, 3, 1))
    Wo_bf = Wo.astype(jnp.bfloat16)

    def body(x_ref, wq_ref, kt_ref, vt_ref, wo_ref, out_ref,
             ctx_ref, send_sems, recv_sems):
        my = lax.axis_index("i")

        x2 = x_ref[...].reshape(BS, D_MODEL)
        q = jnp.dot(x2, wq_ref[...], preferred_element_type=jnp.float32)

        barrier_sem = pltpu.get_barrier_semaphore()
        for off in range(1, N_DEV):
            pl.semaphore_signal(
                barrier_sem, inc=1,
                device_id=((my + off) % N_DEV,),
                device_id_type=pl.DeviceIdType.MESH,
            )
        pl.semaphore_wait(barrier_sem, N_DEV - 1)

        row = lax.broadcasted_iota(jnp.int32, (BS, BS), 0)
        col = lax.broadcasted_iota(jnp.int32, (BS, BS), 1)
        qb = (row % Sq) // 64
        kb = (col % Skv) // 64
        mask = ((row // Sq) == (col // Skv)) & (
            (qb == kb) | (kb == 0) | ((qb + kb) % 3 == 0)
        )
        neg = jnp.float32(-1e9)

        q_bf = q.astype(jnp.bfloat16)
        heads = []
        for h in range(HQ_LOCAL):
            kT = jnp.concatenate(
                [kt_ref[0, h], kt_ref[1, h]], axis=1)
            s = jnp.dot(q_bf[:, h * Dh:(h + 1) * Dh],
                        kT.astype(jnp.bfloat16),
                        preferred_element_type=jnp.float32) * 0.125
            e = jnp.exp(jnp.where(mask, s, neg))
            denom = jnp.sum(e, axis=-1, keepdims=True)
            vT = jnp.concatenate(
                [vt_ref[0, h], vt_ref[1, h]], axis=1)
            ctx_h = lax.dot_general(
                e.astype(jnp.bfloat16), vT.astype(jnp.bfloat16),
                (((1,), (1,)), ((), ())),
                preferred_element_type=jnp.float32)
            heads.append((ctx_h / denom).astype(jnp.bfloat16))
        ctx_ref[0] = jnp.concatenate(heads, axis=1)

        sends = []
        for off in (2, 1, 3):
            dst_slot = N_DEV - off
            rdma = pltpu.make_async_remote_copy(
                src_ref=ctx_ref.at[0],
                dst_ref=ctx_ref.at[dst_slot],
                send_sem=send_sems.at[off - 1],
                recv_sem=recv_sems.at[dst_slot - 1],
                device_id=((my + off) % N_DEV,),
                device_id_type=pl.DeviceIdType.MESH,
            )
            rdma.start()
            sends.append(rdma)

        acc = jnp.dot(ctx_ref[0], wo_ref[pl.ds(my * HD, HD), :],
                      preferred_element_type=jnp.float32)
        for j in (1, 3, 2):
            recv = pltpu.make_async_remote_copy(
                src_ref=ctx_ref.at[j],
                dst_ref=ctx_ref.at[j],
                send_sem=send_sems.at[j - 1],
                recv_sem=recv_sems.at[j - 1],
                device_id=((my + j) % N_DEV,),
                device_id_type=pl.DeviceIdType.MESH,
            )
            recv.wait_recv()
            src_dev = (my + j) % N_DEV
            acc = acc + jnp.dot(ctx_ref[j],
                                wo_ref[pl.ds(src_dev * HD, HD), :],
                                preferred_element_type=jnp.float32)
        out_ref[...] = acc.reshape(B, Sq, D_OUT)

        for rdma in sends:
            rdma.wait_send()

    return pl.pallas_call(
        body,
        out_shape=jax.ShapeDtypeStruct((B, Sq, D_OUT), jnp.float32),
        in_specs=[pl.BlockSpec(memory_space=pltpu.VMEM)] * 5,
        out_specs=pl.BlockSpec(memory_space=pltpu.VMEM),
        scratch_shapes=[
            pltpu.VMEM((N_DEV, BS, HD), jnp.bfloat16),
            pltpu.SemaphoreType.DMA((N_DEV - 1,)),
            pltpu.SemaphoreType.DMA((N_DEV - 1,)),
        ],
        compiler_params=pltpu.CompilerParams(collective_id=0),
    )(x, Wq_sl, Kt, Vt, Wo_bf)
